# baseline (speedup 1.0000x reference)
import jax
import jax.numpy as jnp
from jax import lax
from jax.experimental import pallas as pl
from jax.experimental.pallas import tpu as pltpu
from jax.experimental.pallas import tpu_sc as plsc

ROWS = 128
N = 32768
L = 16
NC = 2
NS = 16
NW = NC * NS
SC_ROWS = 64
CPS = 16
SC_ELEMS = CPS * L
NSUP = N // SC_ELEMS
MAX_NEWTON = 20
NEG = -3.0e38

TC_BLOCK = 8
TC_MAX_NEWTON = 48


def _bmax(v, idx):
    for sh in (8, 4, 2, 1):
        v = jnp.maximum(v, v[jnp.bitwise_xor(idx, sh)])
    return v


def _bsum(v, idx):
    for sh in (8, 4, 2, 1):
        v = v + v[jnp.bitwise_xor(idx, sh)]
    return v


def _newton_step(buf, sclist, nsc, tau, changed, idx16, zv, onev):
    trip = jnp.where(changed == 1, nsc, 0)
    tv = jnp.full((L,), tau, jnp.float32)

    def sc_body(u, acc):
        sa, ka = acc
        base = sclist[u] * SC_ELEMS
        for c in range(CPS):
            v = buf[pl.ds(base + c * L, L)]
            msk = v > tv
            sa = sa + jnp.where(msk, v, zv)
            ka = ka + jnp.where(msk, onev, zv)
        return sa, ka

    sa, ka = lax.fori_loop(0, trip, sc_body, (zv, zv))
    sv = _bsum(sa, idx16)
    kv = jnp.maximum(_bsum(ka, idx16), onev)
    tau_new = ((sv - 1.0) / kv)[0]
    keep = changed == 1
    tau_out = jnp.where(keep, tau_new, tau)
    changed_out = jnp.where(
        jnp.logical_and(keep, tau_new != tau), 1, 0).astype(jnp.int32)
    return tau_out, changed_out


def _compute_row(buf, segmax, sclist, idx16):
    zv = jnp.zeros((L,), jnp.float32)
    onev = jnp.full((L,), 1.0, jnp.float32)

    def sup_body(j, gmax):
        def ch_body(c, m):
            return jnp.maximum(m, buf[pl.ds(j * SC_ELEMS + c * L, L)])

        m = lax.fori_loop(0, CPS, ch_body,
                          jnp.full((L,), NEG, jnp.float32), unroll=CPS)
        segmax[pl.ds(j * L, L)] = m
        return jnp.maximum(gmax, m)

    gmax = lax.fori_loop(0, NSUP, sup_body,
                         jnp.full((L,), NEG, jnp.float32))
    tau0 = _bmax(gmax, idx16)[0] - 1.0

    def filt_body(j, nsc):
        m = segmax[pl.ds(j * L, L)]
        smax = _bmax(m, idx16)[0]
        cond = smax > tau0

        @pl.when(cond)
        def _():
            sclist[nsc] = j

        return nsc + jnp.where(cond, 1, 0)

    nsc = lax.fori_loop(0, NSUP, filt_body, jnp.int32(0))

    tau1, ch1 = _newton_step(buf, sclist, nsc, tau0, jnp.int32(1),
                             idx16, zv, onev)

    def rf_body(u, cnt):
        j = sclist[u]
        m = segmax[pl.ds(j * L, L)]
        smax = _bmax(m, idx16)[0]
        cond = smax > tau1

        @pl.when(cond)
        def _():
            sclist[cnt] = j

        return cnt + jnp.where(cond, 1, 0)

    nsc2 = lax.fori_loop(0, nsc, rf_body, jnp.int32(0))

    def newton_it(t, carry):
        tau, changed = carry
        return _newton_step(buf, sclist, nsc2, tau, changed, idx16, zv, onev)

    tau, _ = lax.fori_loop(0, MAX_NEWTON - 1, newton_it, (tau1, ch1))
    return tau, nsc2


def _write_out(buf, outbuf, sclist, nsc2, tau, zv):
    tvo = jnp.full((L,), tau, jnp.float32)

    def wb(u, _):
        base = sclist[u] * SC_ELEMS
        for c in range(CPS):
            v = buf[pl.ds(base + c * L, L)]
            outbuf[pl.ds(base + c * L, L)] = jnp.maximum(v - tvo, zv)
        return 0

    lax.fori_loop(0, nsc2, wb, 0)


def _zero_sup(outbuf, sclist, nsc2, zv):
    def zb(u, _):
        base = sclist[u] * SC_ELEMS
        for c in range(CPS):
            outbuf[pl.ds(base + c * L, L)] = zv
        return 0

    lax.fori_loop(0, nsc2, zb, 0)


def _make_sc_body(rpw):
    def _sc_body(x_hbm, out_hbm, b0, b1, outbuf, segmax, slA, slB, *sems):
        wid = lax.axis_index("s") * NC + lax.axis_index("c")
        idx16 = lax.iota(jnp.int32, L)
        zv = jnp.zeros((L,), jnp.float32)
        r0 = wid * rpw

        bufs = (b0, b1)
        lists = (slA, slB)
        isems = sems[:rpw]
        osems = sems[rpw:]

        h_in = [None] * rpw
        h_out = [None] * rpw

        h_in[0] = pltpu.async_copy(x_hbm.at[r0 + 0], b0, isems[0])

        def zb0(i, _):
            outbuf[pl.ds(i * L, L)] = zv
            return 0

        lax.fori_loop(0, N // L, zb0, 0, unroll=16)

        if rpw > 1:
            h_in[1] = pltpu.async_copy(x_hbm.at[r0 + 1], b1, isems[1])

        prev_nsc2 = None
        for r in range(rpw):
            h_in[r].wait()
            tau, nsc2 = _compute_row(bufs[r % 2], segmax, lists[r % 2], idx16)
            if r >= 1:
                h_out[r - 1].wait()
                _zero_sup(outbuf, lists[(r - 1) % 2], prev_nsc2, zv)
            _write_out(bufs[r % 2], outbuf, lists[r % 2], nsc2, tau, zv)
            if r + 2 < rpw:
                h_in[r + 2] = pltpu.async_copy(
                    x_hbm.at[r0 + r + 2], bufs[r % 2], isems[r + 2])
            h_out[r] = pltpu.async_copy(outbuf, out_hbm.at[r0 + r], osems[r])
            prev_nsc2 = nsc2
        h_out[rpw - 1].wait()

    return _sc_body


def _sc_sparsemax(x):
    rows = x.shape[0]
    rpw = rows // NW
    mesh = plsc.VectorSubcoreMesh(
        core_axis_name="c", subcore_axis_name="s", num_cores=NC, num_subcores=NS)
    fn = pl.kernel(
        _make_sc_body(rpw),
        out_type=jax.ShapeDtypeStruct((rows, N), jnp.float32),
        mesh=mesh,
        scratch_types=[
            pltpu.VMEM((N,), jnp.float32),
            pltpu.VMEM((N,), jnp.float32),
            pltpu.VMEM((N,), jnp.float32),
            pltpu.VMEM((NSUP * L,), jnp.float32),
            pltpu.SMEM((NSUP,), jnp.int32),
            pltpu.SMEM((NSUP,), jnp.int32),
        ] + [pltpu.SemaphoreType.DMA] * (2 * rpw),
    )
    return fn(x)


def _tc_block(x_ref, o_ref):
    x = x_ref[...]
    m = jnp.max(x, axis=1, keepdims=True)
    tau0 = m - 1.0

    def newton_body(carry):
        i, tau, _ = carry
        mask = x > tau
        k = jnp.sum(mask.astype(jnp.float32), axis=1, keepdims=True)
        s = jnp.sum(jnp.where(mask, x, 0.0), axis=1, keepdims=True)
        k = jnp.maximum(k, 1.0)
        tau_new = (s - 1.0) / k
        changed = jnp.any(tau_new != tau)
        return i + 1, tau_new, changed

    def newton_cond(carry):
        i, _, changed = carry
        return jnp.logical_and(i < TC_MAX_NEWTON, changed)

    _, tau, _ = jax.lax.while_loop(
        newton_cond, newton_body, (jnp.int32(0), tau0, jnp.bool_(True)))
    o_ref[...] = jnp.maximum(x - tau, 0.0)


def _tc_sparsemax(x):
    rows = x.shape[0]
    grid = rows // TC_BLOCK
    return pl.pallas_call(
        _tc_block,
        grid=(grid,),
        in_specs=[pl.BlockSpec((TC_BLOCK, N), lambda i: (i, 0))],
        out_specs=pl.BlockSpec((TC_BLOCK, N), lambda i: (i, 0)),
        out_shape=jax.ShapeDtypeStruct((rows, N), jnp.float32),
    )(x)


def kernel(input):
    sc_out = _sc_sparsemax(input[:SC_ROWS])
    tc_out = _tc_sparsemax(input[SC_ROWS:])
    return jnp.concatenate([sc_out, tc_out], axis=0)
